# baseline (speedup 1.0000x reference)
import jax
import jax.numpy as jnp
from jax import lax
from jax.experimental import pallas as pl
from jax.experimental.pallas import tpu as pltpu
from jax.experimental.pallas import tpu_sc as plsc

_VOCAB = 1000
_EMB = 64
_BATCH = 16384
_HIST = 50

_NC = 2
_NS = 16
_NW = _NC * _NS

_BW = _BATCH // _NW
_HB = _BW // 256
_L = 16


def _body(ids_hbm, table_hbm, out_hbm, idx_v, table_v, tb0, tb1,
          ic0, ic1, isem, tsem, ssem0, ssem1):
    c_id = lax.axis_index("c")
    s_id = lax.axis_index("s")
    wid = s_id * _NC + c_id
    base = wid * _BW

    pltpu.async_copy(ids_hbm.at[pl.ds(base, _BW)], idx_v, isem)
    pltpu.async_copy(table_hbm, table_v, tsem)
    pltpu.make_async_copy(ids_hbm.at[pl.ds(base, _BW)], idx_v, isem).wait()
    pltpu.make_async_copy(table_hbm, table_v, tsem).wait()

    tbufs = (tb0, tb1)
    idcols = (ic0, ic1)
    ssems = (ssem0, ssem1)
    iota = lax.iota(jnp.int32, _L)

    def store_desc(h, half, b):
        return pltpu.make_async_copy(
            tbufs[b],
            out_hbm.at[h, :, pl.ds(4 * wid + 2 * half, 2)],
            ssems[b],
        )

    def do_unit(h, half, b, first):
        hvec = jnp.full((_L,), h, jnp.int32)
        for j in range(16):
            idcols[b][pl.ds(_L * j, _L)] = plsc.load_gather(
                idx_v, [iota + (half * 256 + _L * j), hvec]
            )

        @pl.when(jnp.logical_not(first))
        def _():
            store_desc(h, half, b).wait()

        def col_body(i, carry):
            eb = i // 8
            de = i - eb * 8
            col = jnp.full((_L,), i, jnp.int32)
            for j in range(16):
                bb = j // 8
                db0 = _L * (j - bb * 8)
                idv = idcols[b][pl.ds(_L * j, _L)]
                vals = plsc.load_gather(table_v, [idv, col])
                tbufs[b][eb, bb, de, pl.ds(db0, _L)] = vals
            return carry

        lax.fori_loop(0, _EMB, col_body, 0)
        store_desc(h, half, b).start()

    def h_body(h, carry):
        do_unit(h, 0, 0, h == 0)
        do_unit(h, 1, 1, h == 0)
        return carry

    lax.fori_loop(0, _HIST, h_body, 0)

    store_desc(_HIST - 1, 0, 0).wait()
    store_desc(_HIST - 1, 1, 1).wait()


def kernel(ids, table):
    run = pl.kernel(
        _body,
        out_type=jax.ShapeDtypeStruct((_HIST, 8, 128, 8, 128), jnp.float32),
        mesh=plsc.VectorSubcoreMesh(core_axis_name="c", subcore_axis_name="s"),
        compiler_params=pltpu.CompilerParams(use_tc_tiling_on_sc=False, needs_layout_passes=False),
        scratch_types=[
            pltpu.VMEM((_BW, _HIST), jnp.int32),
            pltpu.VMEM((_VOCAB, _EMB), jnp.float32),
            pltpu.VMEM((8, 2, 8, 128), jnp.float32),
            pltpu.VMEM((8, 2, 8, 128), jnp.float32),
            pltpu.VMEM((256,), jnp.int32),
            pltpu.VMEM((256,), jnp.int32),
            pltpu.SemaphoreType.DMA,
            pltpu.SemaphoreType.DMA,
            pltpu.SemaphoreType.DMA,
            pltpu.SemaphoreType.DMA,
        ],
    )
    out5 = run(ids, table)
    return out5.transpose(2, 4, 0, 1, 3).reshape(_BATCH, _HIST, _EMB)


# speedup vs baseline: 1.5814x; 1.5814x over previous
import functools

import jax
import jax.numpy as jnp
from jax import lax
from jax.experimental import pallas as pl
from jax.experimental.pallas import tpu as pltpu
from jax.experimental.pallas import tpu_sc as plsc

_VOCAB = 1000
_EMB = 64
_BATCH = 16384
_HIST = 50

_NC = 2
_NS = 16
_NW = _NC * _NS

_BW = _BATCH // _NW
_L = 16


def _body(ids_hbm, table_hbm, out_hbm, idsT_v, table_v, tb0, tb1,
          isem, tsem, ssem0, ssem1):
    c_id = lax.axis_index("c")
    s_id = lax.axis_index("s")
    wid = s_id * _NC + c_id
    base = wid * _BW

    pltpu.async_copy(ids_hbm.at[:, pl.ds(base, _BW)], idsT_v, isem)
    pltpu.async_copy(table_hbm, table_v, tsem)
    pltpu.make_async_copy(ids_hbm.at[:, pl.ds(base, _BW)], idsT_v, isem).wait()
    pltpu.make_async_copy(table_hbm, table_v, tsem).wait()

    tbufs = (tb0, tb1)
    ssems = (ssem0, ssem1)

    def store_desc(h, half):
        return pltpu.make_async_copy(
            tbufs[half],
            out_hbm.at[h, :, pl.ds(4 * wid + 2 * half, 2)],
            ssems[half],
        )

    def do_unit(h, half, first):
        idvecs = [
            idsT_v[h, pl.ds(256 * half + _L * j, _L)] for j in range(16)
        ]

        @pl.when(jnp.logical_not(first))
        def _():
            store_desc(h, half).wait()

        def col_body(i, carry):
            eb = i // 8
            de = i - eb * 8
            for j in range(16):
                bb = j // 8
                db0 = _L * (j - bb * 8)
                vals = plsc.load_gather(table_v, [idvecs[j] + i])
                tbufs[half][eb, bb, de, pl.ds(db0, _L)] = vals
            return carry

        lax.fori_loop(0, _EMB, col_body, 0)

        store_desc(h, half).start()

    def h_body(h, carry):
        do_unit(h, 0, h == 0)
        do_unit(h, 1, h == 0)
        return carry

    lax.fori_loop(0, _HIST, h_body, 0)

    store_desc(_HIST - 1, 0).wait()
    store_desc(_HIST - 1, 1).wait()


def kernel(ids, table):
    run = pl.kernel(
        _body,
        out_type=jax.ShapeDtypeStruct((_HIST, 8, 128, 8, 128), jnp.float32),
        mesh=plsc.VectorSubcoreMesh(core_axis_name="c", subcore_axis_name="s"),
        compiler_params=pltpu.CompilerParams(
            use_tc_tiling_on_sc=False, needs_layout_passes=False
        ),
        scratch_types=[
            pltpu.VMEM((_HIST, _BW), jnp.int32),
            pltpu.VMEM((_VOCAB * _EMB,), jnp.float32),
            pltpu.VMEM((8, 2, 8, 128), jnp.float32),
            pltpu.VMEM((8, 2, 8, 128), jnp.float32),
            pltpu.SemaphoreType.DMA,
            pltpu.SemaphoreType.DMA,
            pltpu.SemaphoreType.DMA,
            pltpu.SemaphoreType.DMA,
        ],
    )
    ids64t = (ids * _EMB).T
    out5 = run(ids64t, table.reshape(_VOCAB * _EMB))
    return out5.transpose(2, 4, 0, 1, 3).reshape(_BATCH, _HIST, _EMB)


# speedup vs baseline: 1.9067x; 1.2057x over previous
import functools

import jax
import jax.numpy as jnp
from jax import lax
from jax.experimental import pallas as pl
from jax.experimental.pallas import tpu as pltpu
from jax.experimental.pallas import tpu_sc as plsc

_VOCAB = 1000
_EMB = 64
_BATCH = 16384
_HIST = 50

_NC = 2
_NS = 16
_NW = _NC * _NS

_BW = _BATCH // _NW
_L = 16


def _body(ids_hbm, table_hbm, out_hbm, idsT_v, table_v, tb0, tb1,
          isem, tsem, ssem0, ssem1):
    c_id = lax.axis_index("c")
    s_id = lax.axis_index("s")
    wid = s_id * _NC + c_id
    base = wid * _BW

    pltpu.async_copy(ids_hbm.at[:, pl.ds(base, _BW)], idsT_v, isem)
    pltpu.async_copy(table_hbm, table_v, tsem)
    pltpu.make_async_copy(ids_hbm.at[:, pl.ds(base, _BW)], idsT_v, isem).wait()
    pltpu.make_async_copy(table_hbm, table_v, tsem).wait()

    tbufs = (tb0, tb1)
    ssems = (ssem0, ssem1)

    def store_desc(h, half):
        return pltpu.make_async_copy(
            tbufs[half],
            out_hbm.at[h, :, pl.ds(4 * wid + 2 * half, 2)],
            ssems[half],
        )

    def do_unit(h, half, first):
        idvecs = [
            idsT_v[h, pl.ds(256 * half + _L * j, _L)] for j in range(16)
        ]

        @pl.when(jnp.logical_not(first))
        def _():
            store_desc(h, half).wait()

        def col_body(i, carry):
            eb = i // 8
            de = i - eb * 8
            vals = [
                plsc.load_gather(table_v, [idvecs[j] + i]) for j in range(16)
            ]
            for j in range(16):
                bb = j // 8
                db0 = _L * (j - bb * 8)
                tbufs[half][eb, bb, de, pl.ds(db0, _L)] = vals[j]
            return carry

        lax.fori_loop(0, _EMB, col_body, 0)

        store_desc(h, half).start()

    def h_body(h, carry):
        do_unit(h, 0, h == 0)
        do_unit(h, 1, h == 0)
        return carry

    lax.fori_loop(0, _HIST, h_body, 0)

    store_desc(_HIST - 1, 0).wait()
    store_desc(_HIST - 1, 1).wait()


def kernel(ids, table):
    run = pl.kernel(
        _body,
        out_type=jax.ShapeDtypeStruct((_HIST, 8, 128, 8, 128), jnp.float32),
        mesh=plsc.VectorSubcoreMesh(core_axis_name="c", subcore_axis_name="s"),
        compiler_params=pltpu.CompilerParams(
            use_tc_tiling_on_sc=False, needs_layout_passes=False
        ),
        scratch_types=[
            pltpu.VMEM((_HIST, _BW), jnp.int32),
            pltpu.VMEM((_VOCAB * _EMB,), jnp.float32),
            pltpu.VMEM((8, 2, 8, 128), jnp.float32),
            pltpu.VMEM((8, 2, 8, 128), jnp.float32),
            pltpu.SemaphoreType.DMA,
            pltpu.SemaphoreType.DMA,
            pltpu.SemaphoreType.DMA,
            pltpu.SemaphoreType.DMA,
        ],
    )
    ids64t = (ids * _EMB).T
    out5 = run(ids64t, table.reshape(_VOCAB * _EMB))
    return out5.transpose(2, 4, 0, 1, 3).reshape(_BATCH, _HIST, _EMB)


# speedup vs baseline: 11.4892x; 6.0258x over previous
import jax
import jax.numpy as jnp
from jax import lax
from jax.experimental import pallas as pl
from jax.experimental.pallas import tpu as pltpu
from jax.experimental.pallas import tpu_sc as plsc

_VOCAB = 1000
_EMB = 64
_BATCH = 16384
_HIST = 50

_NC = 2
_NS = 16
_NW = _NC * _NS

_BW = _BATCH // _NW
_L = 16


def _body(ids_hbm, table_hbm, out_hbm, idsT_v, table_v, tb0, tb1,
          isem, tsem, ssem0, ssem1):
    c_id = lax.axis_index("c")
    s_id = lax.axis_index("s")
    wid = s_id * _NC + c_id
    base = wid * _BW

    pltpu.async_copy(ids_hbm.at[:, pl.ds(base, _BW)], idsT_v, isem)
    pltpu.async_copy(table_hbm, table_v, tsem)
    pltpu.make_async_copy(ids_hbm.at[:, pl.ds(base, _BW)], idsT_v, isem).wait()
    pltpu.make_async_copy(table_hbm, table_v, tsem).wait()

    tbufs = (tb0, tb1)
    ssems = (ssem0, ssem1)

    def store_desc(h, half):
        return pltpu.make_async_copy(
            tbufs[half],
            out_hbm.at[h, :, pl.ds(4 * wid + 2 * half, 2)],
            ssems[half],
        )

    def do_unit(h, half, first):
        idvecs = [
            idsT_v[h, pl.ds(256 * half + _L * j, _L)] for j in range(16)
        ]

        @pl.when(jnp.logical_not(first))
        def _():
            store_desc(h, half).wait()

        def col_body(i, carry):
            eb = i // 8
            de = i - eb * 8
            vals = [
                plsc.load_gather(table_v, [idvecs[j] + i]) for j in range(16)
            ]
            for j in range(16):
                bb = j // 8
                db0 = _L * (j - bb * 8)
                tbufs[half][eb, bb, de, pl.ds(db0, _L)] = vals[j]
            return carry

        lax.fori_loop(0, _EMB, col_body, 0)

        store_desc(h, half).start()

    def h_body(h, carry):
        do_unit(h, 0, h == 0)
        do_unit(h, 1, h == 0)
        return carry

    lax.fori_loop(0, _HIST, h_body, 0)

    store_desc(_HIST - 1, 0).wait()
    store_desc(_HIST - 1, 1).wait()


def kernel(ids, table):
    run = pl.kernel(
        _body,
        out_type=jax.ShapeDtypeStruct((_HIST, 8, 128, 8, 128), jnp.float32),
        mesh=plsc.VectorSubcoreMesh(core_axis_name="c", subcore_axis_name="s"),
        compiler_params=pltpu.CompilerParams(
            use_tc_tiling_on_sc=False, needs_layout_passes=False
        ),
        scratch_types=[
            pltpu.VMEM((_HIST, _BW), jnp.int32),
            pltpu.VMEM((_VOCAB * (_EMB + 1),), jnp.float32),
            pltpu.VMEM((8, 2, 8, 128), jnp.float32),
            pltpu.VMEM((8, 2, 8, 128), jnp.float32),
            pltpu.SemaphoreType.DMA,
            pltpu.SemaphoreType.DMA,
            pltpu.SemaphoreType.DMA,
            pltpu.SemaphoreType.DMA,
        ],
    )
    ids65t = (ids * (_EMB + 1)).T
    table65 = jnp.pad(table, ((0, 0), (0, 1))).reshape(_VOCAB * (_EMB + 1))
    out5 = run(ids65t, table65)
    return out5.transpose(2, 4, 0, 1, 3).reshape(_BATCH, _HIST, _EMB)
